# Initial kernel scaffold; baseline (speedup 1.0000x reference)
import jax
import jax.numpy as jnp
from jax.experimental import pallas as pl


def kernel(x, edge_index_0, edge_index_1, edge_index_2, W_0_0, W_0_1, W_1_0, W_1_1, W_2_0, W_2_1, att):
    raise NotImplementedError("write your pallas kernel here")


# speedup vs baseline: 19.9295x; 19.9295x over previous
import functools

import jax
import jax.numpy as jnp
from jax import lax
from jax.experimental import pallas as pl
from jax.experimental.pallas import tpu as pltpu
from jax.experimental.pallas import tpu_sc as plsc

_N = 10000
_NP = 10240
_E = 160000
_EP = 163840
_D = 128
_NC = 2
_NS = 16
_NW = _NC * _NS
_EPW = _EP // _NW
_CH = 128
_NCHUNK = _EPW // _CH
_RPT = _NP // _NS
_BN = 2048
_DW = 128


def _sc_mesh():
    return plsc.VectorSubcoreMesh(core_axis_name="core", subcore_axis_name="sub",
                                  num_cores=_NC, num_subcores=_NS)


@functools.partial(
    pl.kernel,
    out_type=jax.ShapeDtypeStruct((_NC, _NP, _DW), jnp.float32),
    mesh=_sc_mesh(),
    scratch_types=[
        pltpu.VMEM((_NCHUNK, _CH), jnp.int32),
        pltpu.VMEM((_CH, _DW), jnp.float32),
        pltpu.VMEM_SHARED((_NP, _DW), jnp.float32),
    ],
)
def _deg_kernel(d2_0, d2_1, d2_2, onehot, zrows, out, didx, ones_v, acc):
    cid = lax.axis_index("core")
    sid = lax.axis_index("sub")
    wid = cid * _NS + sid
    r0 = sid * _RPT
    pltpu.sync_copy(zrows.at[pl.ds(r0, _RPT), :], acc.at[pl.ds(r0, _RPT), :])
    plsc.subcore_barrier()
    for c in range(3):
        dref = (d2_0, d2_1, d2_2)[c]
        pltpu.sync_copy(onehot.at[c], ones_v)
        pltpu.sync_copy(dref.at[pl.ds(wid * _NCHUNK, _NCHUNK), :], didx)

        def body(j, _):
            pltpu.sync_copy(ones_v, acc.at[didx.at[j]], add=True)
            return ()

        lax.fori_loop(0, _NCHUNK, body, ())
    plsc.subcore_barrier()
    pltpu.sync_copy(acc.at[pl.ds(r0, _RPT), :], out.at[cid, pl.ds(r0, _RPT), :])


@functools.partial(
    pl.kernel,
    out_type=jax.ShapeDtypeStruct((_NC, 3, _NP, _D), jnp.float32),
    mesh=_sc_mesh(),
    scratch_types=[
        pltpu.VMEM((_NCHUNK, _CH), jnp.int32),
        pltpu.VMEM((_NCHUNK, _CH), jnp.int32),
        pltpu.VMEM((_CH, _D), jnp.float32),
        pltpu.VMEM((_CH, _D), jnp.float32),
        pltpu.VMEM_SHARED((_NP, _D), jnp.float32),
        pltpu.SemaphoreType.DMA,
        pltpu.SemaphoreType.DMA,
    ],
)
def _prop_kernel(z0, z1, z2, s2_0, s2_1, s2_2, d2_0, d2_1, d2_2, out,
                 sidx, didx, rows_a, rows_b, acc, sem_a, sem_b):
    cid = lax.axis_index("core")
    sid = lax.axis_index("sub")
    wid = cid * _NS + sid
    r0 = sid * _RPT

    zv = jnp.zeros((16,), jnp.float32)

    for c in range(3):
        zc = (z0, z1, z2)[c]
        sref = (s2_0, s2_1, s2_2)[c]
        dref = (d2_0, d2_1, d2_2)[c]

        @pl.when(cid == 0)
        def _():
            pltpu.sync_copy(zc.at[pl.ds(r0, _RPT), :], acc.at[pl.ds(r0, _RPT), :])

        @pl.when(cid != 0)
        def _():
            def fill_zero(i, _):
                for t in range(_D // 16):
                    rows_a[i, pl.ds(t * 16, 16)] = zv
                return ()

            lax.fori_loop(0, _CH, fill_zero, ())
            for t in range(_RPT // _CH):
                pltpu.sync_copy(rows_a, acc.at[pl.ds(r0 + t * _CH, _CH), :])

        pltpu.sync_copy(sref.at[pl.ds(wid * _NCHUNK, _NCHUNK), :], sidx)
        pltpu.sync_copy(dref.at[pl.ds(wid * _NCHUNK, _NCHUNK), :], didx)
        plsc.subcore_barrier()

        pltpu.async_copy(zc.at[sidx.at[0]], rows_a, sem_a)

        def body(i, _):
            pltpu.make_async_copy(zc.at[sidx.at[0]], rows_a, sem_a).wait()
            pltpu.async_copy(zc.at[sidx.at[2 * i + 1]], rows_b, sem_b)
            pltpu.sync_copy(rows_a, acc.at[didx.at[2 * i]], add=True)
            pltpu.make_async_copy(zc.at[sidx.at[0]], rows_b, sem_b).wait()

            @pl.when(i < _NCHUNK // 2 - 1)
            def _():
                pltpu.async_copy(zc.at[sidx.at[2 * i + 2]], rows_a, sem_a)

            pltpu.sync_copy(rows_b, acc.at[didx.at[2 * i + 1]], add=True)
            return ()

        lax.fori_loop(0, _NCHUNK // 2, body, ())
        plsc.subcore_barrier()
        pltpu.sync_copy(acc.at[pl.ds(r0, _RPT), :], out.at[cid, c, pl.ds(r0, _RPT), :])


def _dinv(g_ref, c):
    dg = g_ref[0, :, c:c + 1] + g_ref[1, :, c:c + 1] + 1.0
    return lax.rsqrt(dg)


def _tc1(x, w0s, deg8):
    def body(x_ref, w_ref, g_ref, z0_ref, z1_ref, z2_ref):
        xb = x_ref[...]
        outs = (z0_ref, z1_ref, z2_ref)
        for c in range(3):
            z = jnp.dot(xb, w_ref[c], preferred_element_type=jnp.float32)
            outs[c][...] = z * _dinv(g_ref, c)

    return pl.pallas_call(
        body,
        grid=(_NP // _BN,),
        in_specs=[
            pl.BlockSpec((_BN, _D), lambda i: (i, 0)),
            pl.BlockSpec((3, _D, _D), lambda i: (0, 0, 0)),
            pl.BlockSpec((_NC, _BN, _DW), lambda i: (0, i, 0)),
        ],
        out_specs=[pl.BlockSpec((_BN, _D), lambda i: (i, 0))] * 3,
        out_shape=[jax.ShapeDtypeStruct((_NP, _D), jnp.float32)] * 3,
    )(x, w0s, deg8)


def _tc2(p, w1s, deg8):
    def body(p_ref, w_ref, g_ref, y0_ref, y1_ref, y2_ref):
        outs = (y0_ref, y1_ref, y2_ref)
        for c in range(3):
            dinv = _dinv(g_ref, c)
            h = jax.nn.relu((p_ref[0, c] + p_ref[1, c]) * dinv)
            y = jnp.dot(h, w_ref[c], preferred_element_type=jnp.float32)
            outs[c][...] = y * dinv

    return pl.pallas_call(
        body,
        grid=(_NP // _BN,),
        in_specs=[
            pl.BlockSpec((_NC, 3, _BN, _D), lambda i: (0, 0, i, 0)),
            pl.BlockSpec((3, _D, _D), lambda i: (0, 0, 0)),
            pl.BlockSpec((_NC, _BN, _DW), lambda i: (0, i, 0)),
        ],
        out_specs=[pl.BlockSpec((_BN, _D), lambda i: (i, 0))] * 3,
        out_shape=[jax.ShapeDtypeStruct((_NP, _D), jnp.float32)] * 3,
    )(p, w1s, deg8)


def _tc3(q, deg8, att):
    def body(q_ref, g_ref, a_ref, o_ref):
        reps, ss = [], []
        for c in range(3):
            rep = (q_ref[0, c] + q_ref[1, c]) * _dinv(g_ref, c)
            reps.append(rep)
            ss.append(jnp.sum(rep * a_ref[:, c, :], axis=1, keepdims=True))
        m = jnp.maximum(jnp.maximum(ss[0], ss[1]), ss[2])
        es = [jnp.exp(s - m) for s in ss]
        den = es[0] + es[1] + es[2]
        o_ref[...] = (es[0] * reps[0] + es[1] * reps[1] + es[2] * reps[2]) / den

    return pl.pallas_call(
        body,
        grid=(_NP // _BN,),
        in_specs=[
            pl.BlockSpec((_NC, 3, _BN, _D), lambda i: (0, 0, i, 0)),
            pl.BlockSpec((_NC, _BN, _DW), lambda i: (0, i, 0)),
            pl.BlockSpec((1, 3, _D), lambda i: (0, 0, 0)),
        ],
        out_specs=pl.BlockSpec((_BN, _D), lambda i: (i, 0)),
        out_shape=jax.ShapeDtypeStruct((_NP, _D), jnp.float32),
    )(q, deg8, att)


def kernel(x, edge_index_0, edge_index_1, edge_index_2,
           W_0_0, W_0_1, W_1_0, W_1_1, W_2_0, W_2_1, att):
    edges = (edge_index_0, edge_index_1, edge_index_2)
    pad_idx = _N + (jnp.arange(_EP - _E, dtype=jnp.int32) % (_NP - _N))
    s2 = [jnp.concatenate([e[0].astype(jnp.int32), pad_idx]).reshape(_EP // _CH, _CH)
          for e in edges]
    d2 = [jnp.concatenate([e[1].astype(jnp.int32), pad_idx]).reshape(_EP // _CH, _CH)
          for e in edges]

    onehot = jnp.broadcast_to(jnp.eye(3, _DW, dtype=jnp.float32)[:, None, :], (3, _CH, _DW))
    zrows = jnp.zeros((_NP, _DW), jnp.float32)
    deg8 = _deg_kernel(d2[0], d2[1], d2[2], onehot, zrows)

    xp = jnp.pad(x, ((0, _NP - _N), (0, 0)))
    w0s = jnp.stack([W_0_0, W_1_0, W_2_0])
    w1s = jnp.stack([W_0_1, W_1_1, W_2_1])

    z0, z1, z2 = _tc1(xp, w0s, deg8)
    p = _prop_kernel(z0, z1, z2, s2[0], s2[1], s2[2], d2[0], d2[1], d2[2])
    y0, y1, y2 = _tc2(p, w1s, deg8)
    q = _prop_kernel(y0, y1, y2, s2[0], s2[1], s2[2], d2[0], d2[1], d2[2])
    return _tc3(q, deg8, att)[:_N]


# speedup vs baseline: 20.6106x; 1.0342x over previous
import functools

import jax
import jax.numpy as jnp
from jax import lax
from jax.experimental import pallas as pl
from jax.experimental.pallas import tpu as pltpu
from jax.experimental.pallas import tpu_sc as plsc

_N = 10000
_NP = 10240
_E = 160000
_EP = 163840
_D = 128
_NC = 2
_NS = 16
_NW = _NC * _NS
_EPW = _EP // _NW
_CH = 128
_NCHUNK = _EPW // _CH
_RPT = _NP // _NS
_BN = 2048


def _sc_mesh():
    return plsc.VectorSubcoreMesh(core_axis_name="core", subcore_axis_name="sub",
                                  num_cores=_NC, num_subcores=_NS)


@functools.partial(
    pl.kernel,
    out_type=jax.ShapeDtypeStruct((_NC, _NP * 8), jnp.float32),
    mesh=_sc_mesh(),
    scratch_types=[
        pltpu.VMEM((_NCHUNK, _CH), jnp.int32),
        pltpu.VMEM((_NCHUNK, _CH), jnp.int32),
        pltpu.VMEM((_CH,), jnp.float32),
        pltpu.VMEM_SHARED((_NP * 8,), jnp.float32),
        pltpu.SemaphoreType.DMA,
    ],
)
def _deg_kernel(d2_0, d2_1, d2_2, zrows, out, didx, fidx, ones_v, acc, sem):
    cid = lax.axis_index("core")
    sid = lax.axis_index("sub")
    wid = cid * _NS + sid
    r0 = sid * _RPT * 8
    onev = jnp.ones((16,), jnp.float32)

    def fill_ones(t, _):
        ones_v[pl.ds(t * 16, 16)] = onev
        return ()

    lax.fori_loop(0, _CH // 16, fill_ones, ())
    pltpu.sync_copy(zrows.at[pl.ds(r0, _RPT * 8)], acc.at[pl.ds(r0, _RPT * 8)])
    plsc.subcore_barrier()
    for c in range(3):
        dref = (d2_0, d2_1, d2_2)[c]
        pltpu.sync_copy(dref.at[pl.ds(wid * _NCHUNK, _NCHUNK), :], didx)

        def flatten(j, _):
            for t in range(_CH // 16):
                v = didx[j, pl.ds(t * 16, 16)]
                fidx[j, pl.ds(t * 16, 16)] = v * 8 + c
            return ()

        lax.fori_loop(0, _NCHUNK, flatten, ())

        def body(j, _):
            pltpu.sync_copy(ones_v, acc.at[fidx.at[j]], add=True)
            return ()

        lax.fori_loop(0, _NCHUNK, body, ())
    plsc.subcore_barrier()
    pltpu.sync_copy(acc.at[pl.ds(r0, _RPT * 8)], out.at[cid, pl.ds(r0, _RPT * 8)])


@functools.partial(
    pl.kernel,
    out_type=jax.ShapeDtypeStruct((_NC, 3, _NP, _D), jnp.float32),
    mesh=_sc_mesh(),
    scratch_types=[
        pltpu.VMEM((_NCHUNK, _CH), jnp.int32),
        pltpu.VMEM((_NCHUNK, _CH), jnp.int32),
        pltpu.VMEM((_CH, _D), jnp.float32),
        pltpu.VMEM((_CH, _D), jnp.float32),
        pltpu.VMEM_SHARED((_NP, _D), jnp.float32),
        pltpu.SemaphoreType.DMA,
        pltpu.SemaphoreType.DMA,
        pltpu.SemaphoreType.DMA,
        pltpu.SemaphoreType.DMA,
    ],
)
def _prop_kernel(z0, z1, z2, s2_0, s2_1, s2_2, d2_0, d2_1, d2_2, out,
                 sidx, didx, rows_a, rows_b, acc, gsem_a, gsem_b, ssem_a, ssem_b):
    cid = lax.axis_index("core")
    sid = lax.axis_index("sub")
    wid = cid * _NS + sid
    r0 = sid * _RPT

    zv = jnp.zeros((16,), jnp.float32)

    for c in range(3):
        zc = (z0, z1, z2)[c]
        sref = (s2_0, s2_1, s2_2)[c]
        dref = (d2_0, d2_1, d2_2)[c]

        @pl.when(cid == 0)
        def _():
            pltpu.sync_copy(zc.at[pl.ds(r0, _RPT), :], acc.at[pl.ds(r0, _RPT), :])

        @pl.when(cid != 0)
        def _():
            def fill_zero(i, _):
                for t in range(_D // 16):
                    rows_a[i, pl.ds(t * 16, 16)] = zv
                return ()

            lax.fori_loop(0, _CH, fill_zero, ())
            for t in range(_RPT // _CH):
                pltpu.sync_copy(rows_a, acc.at[pl.ds(r0 + t * _CH, _CH), :])

        pltpu.sync_copy(sref.at[pl.ds(wid * _NCHUNK, _NCHUNK), :], sidx)
        pltpu.sync_copy(dref.at[pl.ds(wid * _NCHUNK, _NCHUNK), :], didx)
        plsc.subcore_barrier()

        pltpu.async_copy(zc.at[sidx.at[0]], rows_a, gsem_a)
        pltpu.async_copy(zc.at[sidx.at[1]], rows_b, gsem_b)

        def body(i, _):
            pltpu.make_async_copy(zc.at[sidx.at[0]], rows_a, gsem_a).wait()
            pltpu.async_copy(rows_a, acc.at[didx.at[2 * i]], ssem_a, add=True)
            pltpu.make_async_copy(zc.at[sidx.at[0]], rows_b, gsem_b).wait()
            pltpu.async_copy(rows_b, acc.at[didx.at[2 * i + 1]], ssem_b, add=True)
            pltpu.make_async_copy(rows_a, acc.at[didx.at[0]], ssem_a).wait()
            pltpu.make_async_copy(rows_b, acc.at[didx.at[0]], ssem_b).wait()

            @pl.when(i < _NCHUNK // 2 - 1)
            def _():
                pltpu.async_copy(zc.at[sidx.at[2 * i + 2]], rows_a, gsem_a)
                pltpu.async_copy(zc.at[sidx.at[2 * i + 3]], rows_b, gsem_b)

            return ()

        lax.fori_loop(0, _NCHUNK // 2, body, ())
        plsc.subcore_barrier()
        pltpu.sync_copy(acc.at[pl.ds(r0, _RPT), :], out.at[cid, c, pl.ds(r0, _RPT), :])


def _dinv(g_ref, c):
    dg = g_ref[0, :, c:c + 1] + g_ref[1, :, c:c + 1] + 1.0
    return lax.rsqrt(dg)


def _tc1(x, w0s, deg8):
    def body(x_ref, w_ref, g_ref, z0_ref, z1_ref, z2_ref):
        xb = x_ref[...]
        outs = (z0_ref, z1_ref, z2_ref)
        for c in range(3):
            z = jnp.dot(xb, w_ref[c], preferred_element_type=jnp.float32)
            outs[c][...] = z * _dinv(g_ref, c)

    return pl.pallas_call(
        body,
        grid=(_NP // _BN,),
        in_specs=[
            pl.BlockSpec((_BN, _D), lambda i: (i, 0)),
            pl.BlockSpec((3, _D, _D), lambda i: (0, 0, 0)),
            pl.BlockSpec((_NC, _BN, 8), lambda i: (0, i, 0)),
        ],
        out_specs=[pl.BlockSpec((_BN, _D), lambda i: (i, 0))] * 3,
        out_shape=[jax.ShapeDtypeStruct((_NP, _D), jnp.float32)] * 3,
    )(x, w0s, deg8)


def _tc2(p, w1s, deg8):
    def body(p_ref, w_ref, g_ref, y0_ref, y1_ref, y2_ref):
        outs = (y0_ref, y1_ref, y2_ref)
        for c in range(3):
            dinv = _dinv(g_ref, c)
            h = jax.nn.relu((p_ref[0, c] + p_ref[1, c]) * dinv)
            y = jnp.dot(h, w_ref[c], preferred_element_type=jnp.float32)
            outs[c][...] = y * dinv

    return pl.pallas_call(
        body,
        grid=(_NP // _BN,),
        in_specs=[
            pl.BlockSpec((_NC, 3, _BN, _D), lambda i: (0, 0, i, 0)),
            pl.BlockSpec((3, _D, _D), lambda i: (0, 0, 0)),
            pl.BlockSpec((_NC, _BN, 8), lambda i: (0, i, 0)),
        ],
        out_specs=[pl.BlockSpec((_BN, _D), lambda i: (i, 0))] * 3,
        out_shape=[jax.ShapeDtypeStruct((_NP, _D), jnp.float32)] * 3,
    )(p, w1s, deg8)


def _tc3(q, deg8, att):
    def body(q_ref, g_ref, a_ref, o_ref):
        reps, ss = [], []
        for c in range(3):
            rep = (q_ref[0, c] + q_ref[1, c]) * _dinv(g_ref, c)
            reps.append(rep)
            ss.append(jnp.sum(rep * a_ref[:, c, :], axis=1, keepdims=True))
        m = jnp.maximum(jnp.maximum(ss[0], ss[1]), ss[2])
        es = [jnp.exp(s - m) for s in ss]
        den = es[0] + es[1] + es[2]
        o_ref[...] = (es[0] * reps[0] + es[1] * reps[1] + es[2] * reps[2]) / den

    return pl.pallas_call(
        body,
        grid=(_NP // _BN,),
        in_specs=[
            pl.BlockSpec((_NC, 3, _BN, _D), lambda i: (0, 0, i, 0)),
            pl.BlockSpec((_NC, _BN, 8), lambda i: (0, i, 0)),
            pl.BlockSpec((1, 3, _D), lambda i: (0, 0, 0)),
        ],
        out_specs=pl.BlockSpec((_BN, _D), lambda i: (i, 0)),
        out_shape=jax.ShapeDtypeStruct((_NP, _D), jnp.float32),
    )(q, deg8, att)


def kernel(x, edge_index_0, edge_index_1, edge_index_2,
           W_0_0, W_0_1, W_1_0, W_1_1, W_2_0, W_2_1, att):
    edges = (edge_index_0, edge_index_1, edge_index_2)
    pad_idx = _N + (jnp.arange(_EP - _E, dtype=jnp.int32) % (_NP - _N))
    s2 = [jnp.concatenate([e[0].astype(jnp.int32), pad_idx]).reshape(_EP // _CH, _CH)
          for e in edges]
    d2 = [jnp.concatenate([e[1].astype(jnp.int32), pad_idx]).reshape(_EP // _CH, _CH)
          for e in edges]

    zrows = jnp.zeros((_NP * 8,), jnp.float32)
    deg8 = _deg_kernel(d2[0], d2[1], d2[2], zrows).reshape(_NC, _NP, 8)

    xp = jnp.pad(x, ((0, _NP - _N), (0, 0)))
    w0s = jnp.stack([W_0_0, W_1_0, W_2_0])
    w1s = jnp.stack([W_0_1, W_1_1, W_2_1])

    z0, z1, z2 = _tc1(xp, w0s, deg8)
    p = _prop_kernel(z0, z1, z2, s2[0], s2[1], s2[2], d2[0], d2[1], d2[2])
    y0, y1, y2 = _tc2(p, w1s, deg8)
    q = _prop_kernel(y0, y1, y2, s2[0], s2[1], s2[2], d2[0], d2[1], d2[2])
    return _tc3(q, deg8, att)[:_N]


# speedup vs baseline: 25.8104x; 1.2523x over previous
import functools

import jax
import jax.numpy as jnp
from jax import lax
from jax.experimental import pallas as pl
from jax.experimental.pallas import tpu as pltpu
from jax.experimental.pallas import tpu_sc as plsc

_N = 10000
_NP = 10240
_E = 160000
_EP = 163840
_D = 128
_NC = 2
_NS = 16
_NW = _NC * _NS
_EPW = _EP // _NW
_CH = 128
_NCHUNK = _EPW // _CH
_CHP = 64
_NCHP = _EPW // _CHP
_RPT = _NP // _NS
_BN = 2048


def _sc_mesh():
    return plsc.VectorSubcoreMesh(core_axis_name="core", subcore_axis_name="sub",
                                  num_cores=_NC, num_subcores=_NS)


@functools.partial(
    pl.kernel,
    out_type=jax.ShapeDtypeStruct((_NC, _NP * 8), jnp.float32),
    mesh=_sc_mesh(),
    scratch_types=[
        pltpu.VMEM((_NCHUNK, _CH), jnp.int32),
        pltpu.VMEM((_NCHUNK, _CH), jnp.int32),
        pltpu.VMEM((_CH,), jnp.float32),
        pltpu.VMEM_SHARED((_NP * 8,), jnp.float32),
        pltpu.SemaphoreType.DMA,
    ],
)
def _deg_kernel(d2_0, d2_1, d2_2, zrows, out, didx, fidx, ones_v, acc, sem):
    cid = lax.axis_index("core")
    sid = lax.axis_index("sub")
    wid = cid * _NS + sid
    r0 = sid * _RPT * 8
    onev = jnp.ones((16,), jnp.float32)

    def fill_ones(t, _):
        ones_v[pl.ds(t * 16, 16)] = onev
        return ()

    lax.fori_loop(0, _CH // 16, fill_ones, ())
    pltpu.sync_copy(zrows.at[pl.ds(r0, _RPT * 8)], acc.at[pl.ds(r0, _RPT * 8)])
    plsc.subcore_barrier()
    for c in range(3):
        dref = (d2_0, d2_1, d2_2)[c]
        pltpu.sync_copy(dref.at[pl.ds(wid * _NCHUNK, _NCHUNK), :], didx)

        def flatten(j, _):
            for t in range(_CH // 16):
                v = didx[j, pl.ds(t * 16, 16)]
                fidx[j, pl.ds(t * 16, 16)] = v * 8 + c
            return ()

        lax.fori_loop(0, _NCHUNK, flatten, ())

        def body(j, _):
            pltpu.sync_copy(ones_v, acc.at[fidx.at[j]], add=True)
            return ()

        lax.fori_loop(0, _NCHUNK, body, ())
    plsc.subcore_barrier()
    pltpu.sync_copy(acc.at[pl.ds(r0, _RPT * 8)], out.at[cid, pl.ds(r0, _RPT * 8)])


@functools.partial(
    pl.kernel,
    out_type=jax.ShapeDtypeStruct((_NC, 3, _NP, _D), jnp.float32),
    mesh=_sc_mesh(),
    scratch_types=[
        pltpu.VMEM((_NCHP, _CHP), jnp.int32),
        pltpu.VMEM((_NCHP, _CHP), jnp.int32),
        pltpu.VMEM((_CHP, _D), jnp.float32),
        pltpu.VMEM((_CHP, _D), jnp.float32),
        pltpu.VMEM((_CHP, _D), jnp.float32),
        pltpu.VMEM_SHARED((_NP, _D), jnp.float32),
        pltpu.SemaphoreType.DMA,
        pltpu.SemaphoreType.DMA,
        pltpu.SemaphoreType.DMA,
        pltpu.SemaphoreType.DMA,
        pltpu.SemaphoreType.DMA,
        pltpu.SemaphoreType.DMA,
    ],
)
def _prop_kernel(z0, z1, z2, s2_0, s2_1, s2_2, d2_0, d2_1, d2_2, out,
                 sidx, didx, r0b, r1b, r2b, acc,
                 g0, g1, g2, s0, s1, s2):
    cid = lax.axis_index("core")
    sid = lax.axis_index("sub")
    wid = cid * _NS + sid
    r0 = sid * _RPT

    zv = jnp.zeros((16,), jnp.float32)
    rows = (r0b, r1b, r2b)
    gsem = (g0, g1, g2)
    ssem = (s0, s1, s2)

    def wait_gather(b):
        pltpu.make_async_copy(z0.at[sidx.at[0]], rows[b], gsem[b]).wait()

    def wait_scatter(b):
        pltpu.make_async_copy(rows[b], acc.at[didx.at[0]], ssem[b]).wait()

    for c in range(3):
        zc = (z0, z1, z2)[c]
        sref = (s2_0, s2_1, s2_2)[c]
        dref = (d2_0, d2_1, d2_2)[c]

        @pl.when(cid == 0)
        def _():
            pltpu.sync_copy(zc.at[pl.ds(r0, _RPT), :], acc.at[pl.ds(r0, _RPT), :])

        @pl.when(cid != 0)
        def _():
            def fill_zero(i, _):
                for t in range(_D // 16):
                    r0b[i, pl.ds(t * 16, 16)] = zv
                return ()

            lax.fori_loop(0, _CHP, fill_zero, ())
            for t in range(_RPT // _CHP):
                pltpu.sync_copy(r0b, acc.at[pl.ds(r0 + t * _CHP, _CHP), :])

        pltpu.sync_copy(sref.at[pl.ds(wid * _NCHP, _NCHP), :], sidx)
        pltpu.sync_copy(dref.at[pl.ds(wid * _NCHP, _NCHP), :], didx)
        plsc.subcore_barrier()

        pltpu.async_copy(zc.at[sidx.at[0]], rows[0], gsem[0])
        pltpu.async_copy(zc.at[sidx.at[1]], rows[1], gsem[1])

        def slot(i, b, skip_wait=False):
            wait_gather(b)
            pltpu.async_copy(rows[b], acc.at[didx.at[i]], ssem[b], add=True)
            nb = (b + 2) % 3
            if not skip_wait:
                wait_scatter(nb)
            pltpu.async_copy(zc.at[sidx.at[i + 2]], rows[nb], gsem[nb])

        slot(0, 0, skip_wait=True)
        slot(1, 1)
        slot(2, 2)

        def body(j, _):
            slot(3 * j, 0)
            slot(3 * j + 1, 1)
            slot(3 * j + 2, 2)
            return ()

        lax.fori_loop(1, _NCHP // 3, body, ())

        wait_gather(0)
        pltpu.async_copy(rows[0], acc.at[didx.at[_NCHP - 2]], ssem[0], add=True)
        wait_scatter(2)
        wait_gather(1)
        pltpu.async_copy(rows[1], acc.at[didx.at[_NCHP - 1]], ssem[1], add=True)
        wait_scatter(0)
        wait_scatter(1)
        plsc.subcore_barrier()
        pltpu.sync_copy(acc.at[pl.ds(r0, _RPT), :], out.at[cid, c, pl.ds(r0, _RPT), :])


def _dinv(g_ref, c):
    dg = g_ref[0, :, c:c + 1] + g_ref[1, :, c:c + 1] + 1.0
    return lax.rsqrt(dg)


def _tc1(x, w0s, deg8):
    def body(x_ref, w_ref, g_ref, z0_ref, z1_ref, z2_ref):
        xb = x_ref[...]
        outs = (z0_ref, z1_ref, z2_ref)
        for c in range(3):
            z = jnp.dot(xb, w_ref[c], preferred_element_type=jnp.float32)
            outs[c][...] = z * _dinv(g_ref, c)

    return pl.pallas_call(
        body,
        grid=(_NP // _BN,),
        in_specs=[
            pl.BlockSpec((_BN, _D), lambda i: (i, 0)),
            pl.BlockSpec((3, _D, _D), lambda i: (0, 0, 0)),
            pl.BlockSpec((_NC, _BN, 8), lambda i: (0, i, 0)),
        ],
        out_specs=[pl.BlockSpec((_BN, _D), lambda i: (i, 0))] * 3,
        out_shape=[jax.ShapeDtypeStruct((_NP, _D), jnp.float32)] * 3,
    )(x, w0s, deg8)


def _tc2(p, w1s, deg8):
    def body(p_ref, w_ref, g_ref, y0_ref, y1_ref, y2_ref):
        outs = (y0_ref, y1_ref, y2_ref)
        for c in range(3):
            dinv = _dinv(g_ref, c)
            h = jax.nn.relu((p_ref[0, c] + p_ref[1, c]) * dinv)
            y = jnp.dot(h, w_ref[c], preferred_element_type=jnp.float32)
            outs[c][...] = y * dinv

    return pl.pallas_call(
        body,
        grid=(_NP // _BN,),
        in_specs=[
            pl.BlockSpec((_NC, 3, _BN, _D), lambda i: (0, 0, i, 0)),
            pl.BlockSpec((3, _D, _D), lambda i: (0, 0, 0)),
            pl.BlockSpec((_NC, _BN, 8), lambda i: (0, i, 0)),
        ],
        out_specs=[pl.BlockSpec((_BN, _D), lambda i: (i, 0))] * 3,
        out_shape=[jax.ShapeDtypeStruct((_NP, _D), jnp.float32)] * 3,
    )(p, w1s, deg8)


def _tc3(q, deg8, att):
    def body(q_ref, g_ref, a_ref, o_ref):
        reps, ss = [], []
        for c in range(3):
            rep = (q_ref[0, c] + q_ref[1, c]) * _dinv(g_ref, c)
            reps.append(rep)
            ss.append(jnp.sum(rep * a_ref[:, c, :], axis=1, keepdims=True))
        m = jnp.maximum(jnp.maximum(ss[0], ss[1]), ss[2])
        es = [jnp.exp(s - m) for s in ss]
        den = es[0] + es[1] + es[2]
        o_ref[...] = (es[0] * reps[0] + es[1] * reps[1] + es[2] * reps[2]) / den

    return pl.pallas_call(
        body,
        grid=(_NP // _BN,),
        in_specs=[
            pl.BlockSpec((_NC, 3, _BN, _D), lambda i: (0, 0, i, 0)),
            pl.BlockSpec((_NC, _BN, 8), lambda i: (0, i, 0)),
            pl.BlockSpec((1, 3, _D), lambda i: (0, 0, 0)),
        ],
        out_specs=pl.BlockSpec((_BN, _D), lambda i: (i, 0)),
        out_shape=jax.ShapeDtypeStruct((_NP, _D), jnp.float32),
    )(q, deg8, att)


def kernel(x, edge_index_0, edge_index_1, edge_index_2,
           W_0_0, W_0_1, W_1_0, W_1_1, W_2_0, W_2_1, att):
    edges = (edge_index_0, edge_index_1, edge_index_2)
    pad_idx = _N + (jnp.arange(_EP - _E, dtype=jnp.int32) % (_NP - _N))
    s2 = [jnp.concatenate([e[0].astype(jnp.int32), pad_idx]).reshape(_EP // _CH, _CH)
          for e in edges]
    d2 = [jnp.concatenate([e[1].astype(jnp.int32), pad_idx]).reshape(_EP // _CH, _CH)
          for e in edges]

    zrows = jnp.zeros((_NP * 8,), jnp.float32)
    deg8 = _deg_kernel(d2[0], d2[1], d2[2], zrows).reshape(_NC, _NP, 8)

    s2p = [a.reshape(_EP // _CHP, _CHP) for a in s2]
    d2p = [a.reshape(_EP // _CHP, _CHP) for a in d2]

    xp = jnp.pad(x, ((0, _NP - _N), (0, 0)))
    w0s = jnp.stack([W_0_0, W_1_0, W_2_0])
    w1s = jnp.stack([W_0_1, W_1_1, W_2_1])

    z0, z1, z2 = _tc1(xp, w0s, deg8)
    p = _prop_kernel(z0, z1, z2, s2p[0], s2p[1], s2p[2], d2p[0], d2p[1], d2p[2])
    y0, y1, y2 = _tc2(p, w1s, deg8)
    q = _prop_kernel(y0, y1, y2, s2p[0], s2p[1], s2p[2], d2p[0], d2p[1], d2p[2])
    return _tc3(q, deg8, att)[:_N]


# speedup vs baseline: 26.8413x; 1.0399x over previous
import functools

import jax
import jax.numpy as jnp
from jax import lax
from jax.experimental import pallas as pl
from jax.experimental.pallas import tpu as pltpu
from jax.experimental.pallas import tpu_sc as plsc

_N = 10000
_NP = 10240
_E = 160000
_EP = 163840
_D = 128
_NC = 2
_NS = 16
_NW = _NC * _NS
_EPW = _EP // _NW
_CH = 128
_NCHUNK = _EPW // _CH
_CHP = 64
_NCHP = _EPW // _CHP
_RPT = _NP // _NS
_BN = 2048


def _sc_mesh():
    return plsc.VectorSubcoreMesh(core_axis_name="core", subcore_axis_name="sub",
                                  num_cores=_NC, num_subcores=_NS)


@functools.partial(
    pl.kernel,
    out_type=jax.ShapeDtypeStruct((_NC, _NP * 8), jnp.float32),
    mesh=_sc_mesh(),
    scratch_types=[
        pltpu.VMEM((_NCHUNK, _CH), jnp.int32),
        pltpu.VMEM((_NCHUNK, _CH), jnp.int32),
        pltpu.VMEM((_CH,), jnp.float32),
        pltpu.VMEM_SHARED((_NP * 8,), jnp.float32),
        pltpu.SemaphoreType.DMA,
    ],
)
def _deg_kernel(d2_0, d2_1, d2_2, zrows, out, didx, fidx, ones_v, acc, sem):
    cid = lax.axis_index("core")
    sid = lax.axis_index("sub")
    wid = cid * _NS + sid
    r0 = sid * _RPT * 8
    onev = jnp.ones((16,), jnp.float32)

    def fill_ones(t, _):
        ones_v[pl.ds(t * 16, 16)] = onev
        return ()

    lax.fori_loop(0, _CH // 16, fill_ones, ())
    pltpu.sync_copy(zrows.at[pl.ds(r0, _RPT * 8)], acc.at[pl.ds(r0, _RPT * 8)])
    plsc.subcore_barrier()
    for c in range(3):
        dref = (d2_0, d2_1, d2_2)[c]
        pltpu.sync_copy(dref.at[pl.ds(wid * _NCHUNK, _NCHUNK), :], didx)

        def flatten(j, _):
            for t in range(_CH // 16):
                v = didx[j, pl.ds(t * 16, 16)]
                fidx[j, pl.ds(t * 16, 16)] = v * 8 + c
            return ()

        lax.fori_loop(0, _NCHUNK, flatten, ())

        def body(j, _):
            pltpu.sync_copy(ones_v, acc.at[fidx.at[j]], add=True)
            return ()

        lax.fori_loop(0, _NCHUNK, body, ())
    plsc.subcore_barrier()
    pltpu.sync_copy(acc.at[pl.ds(r0, _RPT * 8)], out.at[cid, pl.ds(r0, _RPT * 8)])


@functools.partial(
    pl.kernel,
    out_type=jax.ShapeDtypeStruct((_NC, 3, _NP, _D), jnp.float32),
    mesh=_sc_mesh(),
    scratch_types=[
        pltpu.VMEM((_NCHP, _CHP), jnp.int32),
        pltpu.VMEM((_NCHP, _CHP), jnp.int32),
        pltpu.VMEM((_CHP, _D), jnp.float32),
        pltpu.VMEM((_CHP, _D), jnp.float32),
        pltpu.VMEM((_CHP, _D), jnp.float32),
        pltpu.VMEM_SHARED((_NP, _D), jnp.float32),
        pltpu.SemaphoreType.DMA,
        pltpu.SemaphoreType.DMA,
        pltpu.SemaphoreType.DMA,
        pltpu.SemaphoreType.DMA,
        pltpu.SemaphoreType.DMA,
        pltpu.SemaphoreType.DMA,
    ],
)
def _prop_kernel(z0, z1, z2, s2_0, s2_1, s2_2, d2_0, d2_1, d2_2, out,
                 sidx, didx, r0b, r1b, r2b, acc,
                 g0, g1, g2, s0, s1, s2):
    cid = lax.axis_index("core")
    sid = lax.axis_index("sub")
    wid = cid * _NS + sid
    r0 = sid * _RPT

    zv = jnp.zeros((16,), jnp.float32)
    rows = (r0b, r1b, r2b)
    gsem = (g0, g1, g2)
    ssem = (s0, s1, s2)

    def wait_gather(b):
        pltpu.make_async_copy(z0.at[sidx.at[0]], rows[b], gsem[b]).wait()

    def wait_scatter(b):
        pltpu.make_async_copy(rows[b], acc.at[didx.at[0]], ssem[b]).wait()

    for c in range(3):
        zc = (z0, z1, z2)[c]
        sref = (s2_0, s2_1, s2_2)[c]
        dref = (d2_0, d2_1, d2_2)[c]

        @pl.when(cid == 0)
        def _():
            pltpu.sync_copy(zc.at[pl.ds(r0, _RPT), :], acc.at[pl.ds(r0, _RPT), :])

        @pl.when(cid != 0)
        def _():
            def fill_zero(i, _):
                for t in range(_D // 16):
                    r0b[i, pl.ds(t * 16, 16)] = zv
                return ()

            lax.fori_loop(0, _CHP, fill_zero, ())
            for t in range(_RPT // _CHP):
                pltpu.sync_copy(r0b, acc.at[pl.ds(r0 + t * _CHP, _CHP), :])

        pltpu.sync_copy(sref.at[pl.ds(wid * _NCHP, _NCHP), :], sidx)
        pltpu.sync_copy(dref.at[pl.ds(wid * _NCHP, _NCHP), :], didx)
        plsc.subcore_barrier()

        pltpu.async_copy(zc.at[sidx.at[0]], rows[0], gsem[0])
        pltpu.async_copy(zc.at[sidx.at[1]], rows[1], gsem[1])

        def slot(i, b, skip_wait=False):
            nb = (b + 2) % 3
            if not skip_wait:
                wait_scatter(nb)
            pltpu.async_copy(zc.at[sidx.at[i + 2]], rows[nb], gsem[nb])
            wait_gather(b)
            pltpu.async_copy(rows[b], acc.at[didx.at[i]], ssem[b], add=True)

        slot(0, 0, skip_wait=True)
        slot(1, 1)
        slot(2, 2)

        def body(j, _):
            slot(3 * j, 0)
            slot(3 * j + 1, 1)
            slot(3 * j + 2, 2)
            return ()

        lax.fori_loop(1, _NCHP // 3, body, ())

        wait_gather(0)
        pltpu.async_copy(rows[0], acc.at[didx.at[_NCHP - 2]], ssem[0], add=True)
        wait_scatter(2)
        wait_gather(1)
        pltpu.async_copy(rows[1], acc.at[didx.at[_NCHP - 1]], ssem[1], add=True)
        wait_scatter(0)
        wait_scatter(1)
        plsc.subcore_barrier()
        pltpu.sync_copy(acc.at[pl.ds(r0, _RPT), :], out.at[cid, c, pl.ds(r0, _RPT), :])


def _dinv(g_ref, c):
    dg = g_ref[0, :, c:c + 1] + g_ref[1, :, c:c + 1] + 1.0
    return lax.rsqrt(dg)


def _tc1(x, w0s, deg8):
    def body(x_ref, w_ref, g_ref, z0_ref, z1_ref, z2_ref):
        xb = x_ref[...]
        outs = (z0_ref, z1_ref, z2_ref)
        for c in range(3):
            z = jnp.dot(xb, w_ref[c], preferred_element_type=jnp.float32)
            outs[c][...] = z * _dinv(g_ref, c)

    return pl.pallas_call(
        body,
        grid=(_NP // _BN,),
        in_specs=[
            pl.BlockSpec((_BN, _D), lambda i: (i, 0)),
            pl.BlockSpec((3, _D, _D), lambda i: (0, 0, 0)),
            pl.BlockSpec((_NC, _BN, 8), lambda i: (0, i, 0)),
        ],
        out_specs=[pl.BlockSpec((_BN, _D), lambda i: (i, 0))] * 3,
        out_shape=[jax.ShapeDtypeStruct((_NP, _D), jnp.float32)] * 3,
    )(x, w0s, deg8)


def _tc2(p, w1s, deg8):
    def body(p_ref, w_ref, g_ref, y0_ref, y1_ref, y2_ref):
        outs = (y0_ref, y1_ref, y2_ref)
        for c in range(3):
            dinv = _dinv(g_ref, c)
            h = jax.nn.relu((p_ref[0, c] + p_ref[1, c]) * dinv)
            y = jnp.dot(h, w_ref[c], preferred_element_type=jnp.float32)
            outs[c][...] = y * dinv

    return pl.pallas_call(
        body,
        grid=(_NP // _BN,),
        in_specs=[
            pl.BlockSpec((_NC, 3, _BN, _D), lambda i: (0, 0, i, 0)),
            pl.BlockSpec((3, _D, _D), lambda i: (0, 0, 0)),
            pl.BlockSpec((_NC, _BN, 8), lambda i: (0, i, 0)),
        ],
        out_specs=[pl.BlockSpec((_BN, _D), lambda i: (i, 0))] * 3,
        out_shape=[jax.ShapeDtypeStruct((_NP, _D), jnp.float32)] * 3,
    )(p, w1s, deg8)


def _tc3(q, deg8, att):
    def body(q_ref, g_ref, a_ref, o_ref):
        reps, ss = [], []
        for c in range(3):
            rep = (q_ref[0, c] + q_ref[1, c]) * _dinv(g_ref, c)
            reps.append(rep)
            ss.append(jnp.sum(rep * a_ref[:, c, :], axis=1, keepdims=True))
        m = jnp.maximum(jnp.maximum(ss[0], ss[1]), ss[2])
        es = [jnp.exp(s - m) for s in ss]
        den = es[0] + es[1] + es[2]
        o_ref[...] = (es[0] * reps[0] + es[1] * reps[1] + es[2] * reps[2]) / den

    return pl.pallas_call(
        body,
        grid=(_NP // _BN,),
        in_specs=[
            pl.BlockSpec((_NC, 3, _BN, _D), lambda i: (0, 0, i, 0)),
            pl.BlockSpec((_NC, _BN, 8), lambda i: (0, i, 0)),
            pl.BlockSpec((1, 3, _D), lambda i: (0, 0, 0)),
        ],
        out_specs=pl.BlockSpec((_BN, _D), lambda i: (i, 0)),
        out_shape=jax.ShapeDtypeStruct((_NP, _D), jnp.float32),
    )(q, deg8, att)


def kernel(x, edge_index_0, edge_index_1, edge_index_2,
           W_0_0, W_0_1, W_1_0, W_1_1, W_2_0, W_2_1, att):
    edges = (edge_index_0, edge_index_1, edge_index_2)
    pad_idx = _N + (jnp.arange(_EP - _E, dtype=jnp.int32) % (_NP - _N))
    s2 = [jnp.concatenate([e[0].astype(jnp.int32), pad_idx]).reshape(_EP // _CH, _CH)
          for e in edges]
    d2 = [jnp.concatenate([e[1].astype(jnp.int32), pad_idx]).reshape(_EP // _CH, _CH)
          for e in edges]

    zrows = jnp.zeros((_NP * 8,), jnp.float32)
    deg8 = _deg_kernel(d2[0], d2[1], d2[2], zrows).reshape(_NC, _NP, 8)

    s2p = [a.reshape(_EP // _CHP, _CHP) for a in s2]
    d2p = [a.reshape(_EP // _CHP, _CHP) for a in d2]

    xp = jnp.pad(x, ((0, _NP - _N), (0, 0)))
    w0s = jnp.stack([W_0_0, W_1_0, W_2_0])
    w1s = jnp.stack([W_0_1, W_1_1, W_2_1])

    z0, z1, z2 = _tc1(xp, w0s, deg8)
    p = _prop_kernel(z0, z1, z2, s2p[0], s2p[1], s2p[2], d2p[0], d2p[1], d2p[2])
    y0, y1, y2 = _tc2(p, w1s, deg8)
    q = _prop_kernel(y0, y1, y2, s2p[0], s2p[1], s2p[2], d2p[0], d2p[1], d2p[2])
    return _tc3(q, deg8, att)[:_N]
